# baseline (speedup 1.0000x reference)
import functools

import jax
import jax.numpy as jnp
from jax import lax
from jax.experimental import pallas as pl
from jax.experimental.pallas import tpu as pltpu
from jax.experimental.pallas import tpu_sc as plsc

DIM = 32
B = 16384
NC = 2
NS = 16
NW = NC * NS
B_PER_W = B // NW
CH = 128
N_CH = B_PER_W // CH
N_BUF = 3 * N_CH

_mesh = plsc.VectorSubcoreMesh(core_axis_name="c", subcore_axis_name="s")


@functools.partial(
    pl.kernel,
    mesh=_mesh,
    out_type=(
        jax.ShapeDtypeStruct((B, DIM), jnp.float32),
        jax.ShapeDtypeStruct((B, DIM), jnp.float32),
        jax.ShapeDtypeStruct((B, DIM), jnp.float32),
    ),
    scratch_types=[
        pltpu.VMEM((N_BUF, CH), jnp.int32),
        pltpu.VMEM((N_BUF, CH, DIM), jnp.float32),
        pltpu.SemaphoreType.DMA,
        pltpu.SemaphoreType.DMA,
        pltpu.SemaphoreType.DMA,
    ],
    compiler_params=pltpu.CompilerParams(use_tc_tiling_on_sc=False),
)
def _sc_gather3(user_h, pos_h, neg_h, utab_h, itab_h,
                out_u, out_p, out_n, idx_v, rows_v, sem_i, sem_g, sem_w):
    cid = lax.axis_index("c")
    sid = lax.axis_index("s")
    wid = sid * NC + cid
    base = wid * B_PER_W

    idx_srcs = (user_h, pos_h, neg_h)
    tabs = (utab_h, itab_h, itab_h)
    outs = (out_u, out_p, out_n)

    loads = []
    for g in range(3):
        for j in range(N_CH):
            t = g * N_CH + j
            loads.append(pltpu.async_copy(
                idx_srcs[g].at[pl.ds(base + j * CH, CH)], idx_v.at[t], sem_i))
    for c in loads:
        c.wait()

    gathers = []
    for g in range(3):
        for j in range(N_CH):
            t = g * N_CH + j
            gathers.append(pltpu.async_copy(
                tabs[g].at[idx_v.at[t]], rows_v.at[t], sem_g))
    for c in gathers:
        c.wait()

    writes = []
    for g in range(3):
        for j in range(N_CH):
            t = g * N_CH + j
            writes.append(pltpu.async_copy(
                rows_v.at[t], outs[g].at[pl.ds(base + j * CH, CH)], sem_w))
    for c in writes:
        c.wait()


def kernel(user, pos, neg, user_table, item_table):
    return _sc_gather3(user, pos, neg, user_table, item_table)


# speedup vs baseline: 2.3319x; 2.3319x over previous
import functools

import jax
import jax.numpy as jnp
from jax import lax
from jax.experimental import pallas as pl
from jax.experimental.pallas import tpu as pltpu
from jax.experimental.pallas import tpu_sc as plsc

DIM = 32
B = 16384
NROWS = 1000000
NC = 2
NS = 16
NW = NC * NS
B_PER_W = B // NW
NBUF = 16
NQ = B_PER_W // NBUF

_mesh = plsc.VectorSubcoreMesh(core_axis_name="c", subcore_axis_name="s")


@functools.partial(
    pl.kernel,
    mesh=_mesh,
    out_type=(
        jax.ShapeDtypeStruct((DIM, B), jnp.float32),
        jax.ShapeDtypeStruct((DIM, B), jnp.float32),
        jax.ShapeDtypeStruct((DIM, B), jnp.float32),
    ),
    scratch_types=[
        pltpu.VMEM((B_PER_W,), jnp.int32),
        pltpu.VMEM((B_PER_W,), jnp.int32),
        pltpu.VMEM((B_PER_W,), jnp.int32),
        pltpu.VMEM((NBUF, DIM, 128), jnp.float32),
        pltpu.VMEM((DIM, B_PER_W), jnp.float32),
        pltpu.SemaphoreType.DMA((NBUF,)),
    ],
    compiler_params=pltpu.CompilerParams(needs_layout_passes=False),
)
def _sc_gather3(user_h, pos_h, neg_h, tabTu_h, tabTi_h,
                outTu, outTp, outTn, idx_u, idx_p, idx_n, ring_v, stage_v, sems):
    cid = lax.axis_index("c")
    sid = lax.axis_index("s")
    wid = sid * NC + cid
    base = wid * B_PER_W

    idx_srcs = (user_h, pos_h, neg_h)
    idx_bufs = (idx_u, idx_p, idx_n)
    tabs = (tabTu_h, tabTi_h, tabTi_h)
    outs = (outTu, outTp, outTn)

    for g in range(3):
        pltpu.sync_copy(idx_srcs[g].at[pl.ds(base, B_PER_W)], idx_bufs[g])

    d_lo = lax.iota(jnp.int32, 16)
    d_hi = d_lo + 16

    def _issue(tab, ix, j):
        jcol = pl.multiple_of((ix >> 7) << 7, 128)
        pltpu.async_copy(tab.at[:, pl.ds(jcol, 128)], ring_v.at[j], sems.at[j])

    def _wait(j):
        pltpu.make_async_copy(
            tabs[0].at[:, pl.ds(0, 128)], ring_v.at[j], sems.at[j]
        ).wait()

    def _extract(vec, j, k):
        ix = vec[j]
        lane = jnp.broadcast_to(ix & 127, (16,))
        pos = jnp.broadcast_to(k, (16,))
        v_lo = plsc.load_gather(ring_v.at[j], [d_lo, lane])
        v_hi = plsc.load_gather(ring_v.at[j], [d_hi, lane])
        plsc.store_scatter(stage_v, [d_lo, pos], v_lo)
        plsc.store_scatter(stage_v, [d_hi, pos], v_hi)

    for g in range(3):
        tab = tabs[g]

        idx_buf = idx_bufs[g]

        def body(q, prev_vec, tab=tab, idx_buf=idx_buf):
            vec = idx_buf[pl.ds(q * NBUF, NBUF)]

            @pl.when(q > 0)
            def _():
                for j in range(NBUF):
                    _wait(j)
                    _extract(prev_vec, j, (q - 1) * NBUF + j)

            for j in range(NBUF):
                _issue(tab, vec[j], j)
            return vec

        last_vec = lax.fori_loop(0, NQ, body, jnp.zeros((NBUF,), jnp.int32))
        for j in range(NBUF):
            _wait(j)
            _extract(last_vec, j, (NQ - 1) * NBUF + j)

        pltpu.sync_copy(stage_v, outs[g].at[:, pl.ds(base, B_PER_W)])


def kernel(user, pos, neg, user_table, item_table):
    tabTu = jnp.swapaxes(user_table, 0, 1)
    tabTi = jnp.swapaxes(item_table, 0, 1)
    outTu, outTp, outTn = _sc_gather3(user, pos, neg, tabTu, tabTi)
    return (
        jnp.swapaxes(outTu, 0, 1),
        jnp.swapaxes(outTp, 0, 1),
        jnp.swapaxes(outTn, 0, 1),
    )
